# Initial kernel scaffold; baseline (speedup 1.0000x reference)
import jax
import jax.numpy as jnp
from jax.experimental import pallas as pl


def kernel(x, table):
    raise NotImplementedError("write your pallas kernel here")


# speedup vs baseline: 2.5147x; 2.5147x over previous
import functools
import math

import jax
import jax.numpy as jnp
from jax import lax
from jax.experimental import pallas as pl
from jax.experimental.pallas import tpu as pltpu
from jax.experimental.pallas import tpu_sc as plsc

D = 128
SCALE = math.sqrt(D)
NC = 2
NS = 16
NW = NC * NS
CHUNK = 128
LANES = 16


def _make_emb_kernel(n_chunks: int, vocab: int):
    mesh = plsc.VectorSubcoreMesh(core_axis_name="c", subcore_axis_name="s")

    @functools.partial(
        pl.kernel,
        mesh=mesh,
        out_type=jax.ShapeDtypeStruct((NW * n_chunks, CHUNK, D), jnp.float32),
        scratch_types=[
            pltpu.VMEM((n_chunks, CHUNK), jnp.int32),
            pltpu.VMEM((CHUNK, D), jnp.float32),
            pltpu.SemaphoreType.DMA,
        ],
    )
    def emb(idx_hbm, table_hbm, out_hbm, idx_v, rows_v, gsem):
        c = lax.axis_index("c")
        s = lax.axis_index("s")
        wid = s * NC + c
        pltpu.sync_copy(idx_hbm.at[wid], idx_v)

        def chunk_body(g, carry):
            pltpu.async_copy(table_hbm.at[idx_v.at[g]], rows_v, gsem).wait()

            def scale_row(r, carry2):
                for col in range(D // LANES):
                    sl = pl.ds(col * LANES, LANES)
                    rows_v[r, sl] = rows_v[r, sl] * SCALE
                return carry2

            lax.fori_loop(0, CHUNK, scale_row, 0)

            pltpu.sync_copy(rows_v, out_hbm.at[wid * n_chunks + g])
            return carry

        lax.fori_loop(0, n_chunks, chunk_body, 0)

    return emb


@jax.jit
def kernel(x, table):
    b, h = x.shape
    n_total = b * h
    assert n_total % (NW * CHUNK) == 0
    n_chunks = n_total // (NW * CHUNK)
    idx = x.reshape(NW, n_chunks, CHUNK).astype(jnp.int32)
    emb = _make_emb_kernel(n_chunks, table.shape[0])
    out = emb(idx, table)
    return out.reshape(b, h, D)


# speedup vs baseline: 3.0529x; 1.2140x over previous
import functools
import math

import jax
import jax.numpy as jnp
from jax import lax
from jax.experimental import pallas as pl
from jax.experimental.pallas import tpu as pltpu
from jax.experimental.pallas import tpu_sc as plsc

D = 128
SCALE = math.sqrt(D)
NC = 2
NS = 16
NW = NC * NS
CHUNK = 128
LANES = 16
NBUF = 2


def _make_emb_kernel(n_chunks: int):
    assert n_chunks % NBUF == 0 and n_chunks // NBUF >= 2
    n_groups = n_chunks // NBUF
    mesh = plsc.VectorSubcoreMesh(core_axis_name="c", subcore_axis_name="s")

    @functools.partial(
        pl.kernel,
        mesh=mesh,
        out_type=jax.ShapeDtypeStruct((NW * n_chunks, CHUNK, D), jnp.float32),
        scratch_types=[
            pltpu.VMEM((n_chunks, CHUNK), jnp.int32),
            pltpu.VMEM((NBUF, CHUNK, D), jnp.float32),
            pltpu.VMEM((NBUF, CHUNK, D), jnp.float32),
            pltpu.SemaphoreType.DMA,
            pltpu.SemaphoreType.DMA,
            pltpu.SemaphoreType.DMA,
            pltpu.SemaphoreType.DMA,
        ],
    )
    def emb(idx_hbm, table_hbm, out_hbm, idx_v, in_v, out_v, g0, g1, o0, o1):
        gsems = (g0, g1)
        osems = (o0, o1)
        c = lax.axis_index("c")
        s = lax.axis_index("s")
        wid = s * NC + c
        pltpu.sync_copy(idx_hbm.at[wid], idx_v)

        def scale(b):
            def scale_row(r, carry):
                for col in range(D // LANES):
                    sl = pl.ds(col * LANES, LANES)
                    out_v[b, r, sl] = in_v[b, r, sl] * SCALE
                return carry

            lax.fori_loop(0, CHUNK, scale_row, 0)

        def chunk_step(g, b, do_owait, do_prefetch):
            pltpu.make_async_copy(
                table_hbm.at[idx_v.at[g]], in_v.at[b], gsems[b]
            ).wait()
            if do_owait:
                pltpu.make_async_copy(out_v.at[b], out_hbm.at[0], osems[b]).wait()
            scale(b)
            if do_prefetch:
                pltpu.async_copy(
                    table_hbm.at[idx_v.at[g + NBUF]], in_v.at[b], gsems[b]
                )
            pltpu.async_copy(out_v.at[b], out_hbm.at[wid * n_chunks + g], osems[b])

        for b in range(NBUF):
            pltpu.async_copy(table_hbm.at[idx_v.at[b]], in_v.at[b], gsems[b])

        for b in range(NBUF):
            chunk_step(b, b, do_owait=False, do_prefetch=True)

        def group(t, carry):
            for b in range(NBUF):
                chunk_step(t * NBUF + b, b, do_owait=True, do_prefetch=True)
            return carry

        lax.fori_loop(1, n_groups - 1, group, 0)

        for b in range(NBUF):
            chunk_step((n_groups - 1) * NBUF + b, b, do_owait=True,
                       do_prefetch=False)

        for b in range(NBUF):
            pltpu.make_async_copy(out_v.at[b], out_hbm.at[0], osems[b]).wait()

    return emb


@jax.jit
def kernel(x, table):
    b, h = x.shape
    n_total = b * h
    assert n_total % (NW * CHUNK) == 0
    n_chunks = n_total // (NW * CHUNK)
    idx = x.reshape(NW, n_chunks, CHUNK).astype(jnp.int32)
    emb = _make_emb_kernel(n_chunks)
    out = emb(idx, table)
    return out.reshape(b, h, D)
